# Initial kernel scaffold; baseline (speedup 1.0000x reference)
import jax
import jax.numpy as jnp
from jax.experimental import pallas as pl


def kernel(edge_index, shape_features, W1, b1, W2, b2):
    raise NotImplementedError("write your pallas kernel here")


# speedup vs baseline: 7.6505x; 7.6505x over previous
import functools

import jax
import jax.numpy as jnp
from jax import lax
from jax.experimental import pallas as pl
from jax.experimental.pallas import tpu as pltpu
from jax.experimental.pallas import tpu_sc as plsc

_N = 10000
_E = 320000
_D = 128
_NC = 2
_NS = 16
_NW = _NC * _NS
_C = 80
_EPW = _E // _NW
_CH = _EPW // _C
_NP = 10240
_RPT = _NP // _NS

_mesh = plsc.VectorSubcoreMesh(
    core_axis_name="c", subcore_axis_name="s", num_cores=_NC, num_subcores=_NS)


@functools.partial(
    pl.kernel,
    out_type=jax.ShapeDtypeStruct((_NC, _NP, _D), jnp.float32),
    mesh=_mesh,
    scratch_types=[
        pltpu.VMEM((_CH, _C), jnp.int32),
        pltpu.VMEM((_CH, _C), jnp.int32),
        pltpu.VMEM((_C, _D), jnp.float32),
        pltpu.VMEM_SHARED((_NP, _D), jnp.float32),
        pltpu.SemaphoreType.DMA,
    ],
)
def _sc_aggregate(src_hbm, dst_hbm, x_hbm, zeros_hbm, part_hbm,
                  src_v, dst_v, rows_v, aggr_sh, sem):
    c = lax.axis_index("c")
    s = lax.axis_index("s")
    w = s * _NC + c

    pltpu.sync_copy(zeros_hbm.at[pl.ds(s * _RPT, _RPT)],
                    aggr_sh.at[pl.ds(s * _RPT, _RPT)])

    pltpu.sync_copy(src_hbm.at[w], src_v)
    pltpu.sync_copy(dst_hbm.at[w], dst_v)

    plsc.subcore_barrier()

    def _chunk(k, carry):
        pltpu.async_copy(x_hbm.at[src_v.at[k]], rows_v, sem).wait()
        pltpu.sync_copy(rows_v, aggr_sh.at[dst_v.at[k]], add=True)
        return carry

    lax.fori_loop(0, _CH, _chunk, 0)

    plsc.subcore_barrier()

    pltpu.sync_copy(aggr_sh.at[pl.ds(s * _RPT, _RPT)],
                    part_hbm.at[c, pl.ds(s * _RPT, _RPT)])


def _tc_body(x_ref, p_ref, w1_ref, w2_ref, b1_ref, b2_ref, o_ref):
    cdims = (((1,), (1,)), ((), ()))
    y = lax.dot_general(x_ref[...], w1_ref[...], cdims,
                        preferred_element_type=jnp.float32)
    aggr = p_ref[0] + p_ref[1]
    y = y + lax.dot_general(aggr, w2_ref[...], cdims,
                            preferred_element_type=jnp.float32)
    o_ref[...] = y + b1_ref[...] + b2_ref[...]


_R = 2000


def _tc_combine(x, partials, W1, W2, b1_2d, b2_2d):
    grid = (_N // _R,)
    return pl.pallas_call(
        _tc_body,
        out_shape=jax.ShapeDtypeStruct((_N, _D), jnp.float32),
        grid=grid,
        in_specs=[
            pl.BlockSpec((_R, _D), lambda i: (i, 0)),
            pl.BlockSpec((_NC, _R, _D), lambda i: (0, i, 0)),
            pl.BlockSpec((_D, _D), lambda i: (0, 0)),
            pl.BlockSpec((_D, _D), lambda i: (0, 0)),
            pl.BlockSpec((1, _D), lambda i: (0, 0)),
            pl.BlockSpec((1, _D), lambda i: (0, 0)),
        ],
        out_specs=pl.BlockSpec((_R, _D), lambda i: (i, 0)),
    )(x, partials, W1, W2, b1_2d, b2_2d)


def kernel(edge_index, shape_features, W1, b1, W2, b2):
    src3d = edge_index[0].reshape(_NW, _CH, _C)
    dst3d = edge_index[1].reshape(_NW, _CH, _C)
    zeros = jnp.zeros((_NP, _D), jnp.float32)
    partials = _sc_aggregate(src3d, dst3d, shape_features, zeros)
    return _tc_combine(shape_features, partials, W1, W2,
                       b1.reshape(1, _D), b2.reshape(1, _D))


# speedup vs baseline: 11.1671x; 1.4596x over previous
import functools

import jax
import jax.numpy as jnp
from jax import lax
from jax.experimental import pallas as pl
from jax.experimental.pallas import tpu as pltpu
from jax.experimental.pallas import tpu_sc as plsc

_N = 10000
_E = 320000
_D = 128
_NC = 2
_NS = 16
_NW = _NC * _NS
_C = 80
_EPW = _E // _NW
_CH = _EPW // _C
_SB = 5
_SCH = _CH // _SB
_NP = 10240
_RPT = _NP // _NS

_mesh = plsc.VectorSubcoreMesh(
    core_axis_name="c", subcore_axis_name="s", num_cores=_NC, num_subcores=_NS)


@functools.partial(
    pl.kernel,
    out_type=jax.ShapeDtypeStruct((_NC, _NP, _D), jnp.float32),
    mesh=_mesh,
    scratch_types=[
        pltpu.VMEM((_SCH, _C), jnp.int32),
        pltpu.VMEM((_SCH, _C), jnp.int32),
        pltpu.VMEM((_C, _D), jnp.float32),
        pltpu.VMEM((_C, _D), jnp.float32),
        pltpu.VMEM_SHARED((_NP, _D), jnp.float32),
        pltpu.SemaphoreType.DMA,
        pltpu.SemaphoreType.DMA,
    ],
)
def _sc_aggregate(src_hbm, dst_hbm, x_hbm, zeros_hbm, part_hbm,
                  src_v, dst_v, rows0, rows1, aggr_sh, sem0, sem1):
    c = lax.axis_index("c")
    s = lax.axis_index("s")
    w = s * _NC + c

    pltpu.sync_copy(zeros_hbm.at[pl.ds(s * _RPT, _RPT)],
                    aggr_sh.at[pl.ds(s * _RPT, _RPT)])

    plsc.subcore_barrier()

    def _superblock(sb, carry):
        pltpu.sync_copy(src_hbm.at[w, sb], src_v)
        pltpu.sync_copy(dst_hbm.at[w, sb], dst_v)
        pltpu.async_copy(x_hbm.at[src_v.at[0]], rows0, sem0)

        def _pair(i, cc):
            k = 2 * i
            pltpu.async_copy(x_hbm.at[src_v.at[k + 1]], rows1, sem1)
            pltpu.make_async_copy(x_hbm.at[src_v.at[k]], rows0, sem0).wait()
            pltpu.sync_copy(rows0, aggr_sh.at[dst_v.at[k]], add=True)
            pltpu.async_copy(x_hbm.at[src_v.at[k + 2]], rows0, sem0)
            pltpu.make_async_copy(x_hbm.at[src_v.at[k + 1]], rows1, sem1).wait()
            pltpu.sync_copy(rows1, aggr_sh.at[dst_v.at[k + 1]], add=True)
            return cc

        lax.fori_loop(0, (_SCH - 1) // 2, _pair, 0)

        pltpu.make_async_copy(x_hbm.at[src_v.at[_SCH - 1]], rows0, sem0).wait()
        pltpu.sync_copy(rows0, aggr_sh.at[dst_v.at[_SCH - 1]], add=True)
        return carry

    lax.fori_loop(0, _SB, _superblock, 0)

    plsc.subcore_barrier()

    pltpu.sync_copy(aggr_sh.at[pl.ds(s * _RPT, _RPT)],
                    part_hbm.at[c, pl.ds(s * _RPT, _RPT)])


def _tc_body(x_ref, p_ref, w1_ref, w2_ref, b1_ref, b2_ref, o_ref):
    cdims = (((1,), (1,)), ((), ()))
    y = lax.dot_general(x_ref[...], w1_ref[...], cdims,
                        preferred_element_type=jnp.float32)
    aggr = p_ref[0] + p_ref[1]
    y = y + lax.dot_general(aggr, w2_ref[...], cdims,
                            preferred_element_type=jnp.float32)
    o_ref[...] = y + b1_ref[...] + b2_ref[...]


_R = 2000


def _tc_combine(x, partials, W1, W2, b1_2d, b2_2d):
    grid = (_N // _R,)
    return pl.pallas_call(
        _tc_body,
        out_shape=jax.ShapeDtypeStruct((_N, _D), jnp.float32),
        grid=grid,
        in_specs=[
            pl.BlockSpec((_R, _D), lambda i: (i, 0)),
            pl.BlockSpec((_NC, _R, _D), lambda i: (0, i, 0)),
            pl.BlockSpec((_D, _D), lambda i: (0, 0)),
            pl.BlockSpec((_D, _D), lambda i: (0, 0)),
            pl.BlockSpec((1, _D), lambda i: (0, 0)),
            pl.BlockSpec((1, _D), lambda i: (0, 0)),
        ],
        out_specs=pl.BlockSpec((_R, _D), lambda i: (i, 0)),
    )(x, partials, W1, W2, b1_2d, b2_2d)


def kernel(edge_index, shape_features, W1, b1, W2, b2):
    src3d = edge_index[0].reshape(_NW, _SB, _SCH, _C)
    dst3d = edge_index[1].reshape(_NW, _SB, _SCH, _C)
    zeros = jnp.zeros((_NP, _D), jnp.float32)
    partials = _sc_aggregate(src3d, dst3d, shape_features, zeros)
    return _tc_combine(shape_features, partials, W1, W2,
                       b1.reshape(1, _D), b2.reshape(1, _D))


# speedup vs baseline: 12.4011x; 1.1105x over previous
import functools

import jax
import jax.numpy as jnp
from jax import lax
from jax.experimental import pallas as pl
from jax.experimental.pallas import tpu as pltpu
from jax.experimental.pallas import tpu_sc as plsc

_N = 10000
_E = 320000
_D = 128
_NC = 2
_NS = 16
_NW = _NC * _NS
_C = 80
_EPW = _E // _NW
_CH = _EPW // _C
_SB = 5
_SCH = _CH // _SB
_NP = 10240
_RPT = _NP // _NS

_mesh = plsc.VectorSubcoreMesh(
    core_axis_name="c", subcore_axis_name="s", num_cores=_NC, num_subcores=_NS)


@functools.partial(
    pl.kernel,
    out_type=jax.ShapeDtypeStruct((_NC, _NP, _D), jnp.float32),
    mesh=_mesh,
    scratch_types=[
        pltpu.VMEM((_SCH, _C), jnp.int32),
        pltpu.VMEM((_SCH, _C), jnp.int32),
        pltpu.VMEM((_C, _D), jnp.float32),
        pltpu.VMEM((_C, _D), jnp.float32),
        pltpu.VMEM((_C, _D), jnp.float32),
        pltpu.VMEM_SHARED((_NP, _D), jnp.float32),
        pltpu.SemaphoreType.DMA,
        pltpu.SemaphoreType.DMA,
        pltpu.SemaphoreType.DMA,
        pltpu.SemaphoreType.DMA,
        pltpu.SemaphoreType.DMA,
        pltpu.SemaphoreType.DMA,
    ],
)
def _sc_aggregate(src_hbm, dst_hbm, x_hbm, zeros_hbm, part_hbm,
                  src_v, dst_v, rows0, rows1, rows2, aggr_sh,
                  g0, g1, g2, s0, s1, s2):
    c = lax.axis_index("c")
    s = lax.axis_index("s")
    w = s * _NC + c
    rows = (rows0, rows1, rows2)
    gsem = (g0, g1, g2)
    ssem = (s0, s1, s2)

    pltpu.sync_copy(zeros_hbm.at[pl.ds(s * _RPT, _RPT)],
                    aggr_sh.at[pl.ds(s * _RPT, _RPT)])

    plsc.subcore_barrier()

    def _gather(k, slot):
        pltpu.async_copy(x_hbm.at[src_v.at[k]], rows[slot], gsem[slot])

    def _wait_scatter(k, slot):
        pltpu.make_async_copy(
            rows[slot], aggr_sh.at[dst_v.at[k]], ssem[slot]).wait()

    def _step(k, slot, do_ws, do_g):
        pltpu.make_async_copy(x_hbm.at[src_v.at[k]], rows[slot],
                              gsem[slot]).wait()
        pltpu.async_copy(rows[slot], aggr_sh.at[dst_v.at[k]], ssem[slot],
                         add=True)
        if do_ws:
            _wait_scatter(k - 1, (slot + 2) % 3)
        if do_g:
            _gather(k + 2, (slot + 2) % 3)

    for sb in range(_SB):
        pltpu.sync_copy(src_hbm.at[w, sb], src_v)
        pltpu.sync_copy(dst_hbm.at[w, sb], dst_v)
        _gather(0, 0)
        _gather(1, 1)
        _step(0, 0, do_ws=False, do_g=True)
        _step(1, 1, do_ws=True, do_g=True)

        def _trip(t, cc):
            k = 2 + 3 * t
            _step(k, 2, do_ws=True, do_g=True)
            _step(k + 1, 0, do_ws=True, do_g=True)
            _step(k + 2, 1, do_ws=True, do_g=True)
            return cc

        lax.fori_loop(0, (_SCH - 4) // 3, _trip, 0)

        _step(_SCH - 2, 2, do_ws=True, do_g=False)
        _step(_SCH - 1, 0, do_ws=True, do_g=False)
        _wait_scatter(_SCH - 1, 0)

    plsc.subcore_barrier()

    pltpu.sync_copy(aggr_sh.at[pl.ds(s * _RPT, _RPT)],
                    part_hbm.at[c, pl.ds(s * _RPT, _RPT)])


def _tc_body(x_ref, p_ref, w1_ref, w2_ref, b1_ref, b2_ref, o_ref):
    cdims = (((1,), (1,)), ((), ()))
    y = lax.dot_general(x_ref[...], w1_ref[...], cdims,
                        preferred_element_type=jnp.float32)
    aggr = p_ref[0] + p_ref[1]
    y = y + lax.dot_general(aggr, w2_ref[...], cdims,
                            preferred_element_type=jnp.float32)
    o_ref[...] = y + b1_ref[...] + b2_ref[...]


_R = 2000


def _tc_combine(x, partials, W1, W2, b1_2d, b2_2d):
    grid = (_N // _R,)
    return pl.pallas_call(
        _tc_body,
        out_shape=jax.ShapeDtypeStruct((_N, _D), jnp.float32),
        grid=grid,
        in_specs=[
            pl.BlockSpec((_R, _D), lambda i: (i, 0)),
            pl.BlockSpec((_NC, _R, _D), lambda i: (0, i, 0)),
            pl.BlockSpec((_D, _D), lambda i: (0, 0)),
            pl.BlockSpec((_D, _D), lambda i: (0, 0)),
            pl.BlockSpec((1, _D), lambda i: (0, 0)),
            pl.BlockSpec((1, _D), lambda i: (0, 0)),
        ],
        out_specs=pl.BlockSpec((_R, _D), lambda i: (i, 0)),
    )(x, partials, W1, W2, b1_2d, b2_2d)


def kernel(edge_index, shape_features, W1, b1, W2, b2):
    src3d = edge_index[0].reshape(_NW, _SB, _SCH, _C)
    dst3d = edge_index[1].reshape(_NW, _SB, _SCH, _C)
    zeros = jnp.zeros((_NP, _D), jnp.float32)
    partials = _sc_aggregate(src3d, dst3d, shape_features, zeros)
    return _tc_combine(shape_features, partials, W1, W2,
                       b1.reshape(1, _D), b2.reshape(1, _D))
